# baseline (speedup 1.0000x reference)
import functools

import jax
import jax.numpy as jnp
from jax import lax
from jax.experimental import pallas as pl
from jax.experimental.pallas import tpu as pltpu
from jax.experimental.pallas import tpu_sc as plsc

_B = 16384
_N = 1000
_NPAD = 1024

_R_TC = 0

_BRH = 1024

_NC = 2
_NS = 16
_NW = _NC * _NS
_R_SC = _B - _R_TC
_RPW = _R_SC // _NW
_CH = 16
_NG = _RPW // _CH
_NFULL = _N // 16
_TBASE = _N - 16


def _tau_rows(xv, tab):
    m = jnp.max(xv, axis=1, keepdims=True)
    cols = jax.lax.broadcasted_iota(jnp.int32, xv.shape, 1)
    idx = jnp.min(jnp.where(xv == m, cols, _N), axis=1)
    onehot = cols == idx[:, None]
    return jnp.sum(jnp.where(onehot, tab, 0.0), axis=1)


def _tc_body(xa_ref, xb_ref, lt_ref, o_ref):
    tab = jnp.exp(lt_ref[...])
    o_ref[:_BRH, :] = _tau_rows(xa_ref[...], tab)[:, None]
    o_ref[_BRH:, :] = _tau_rows(xb_ref[...], tab)[:, None]


def _tc_part(x, lt2):
    return pl.pallas_call(
        _tc_body,
        grid=(_R_TC // (2 * _BRH),),
        in_specs=[
            pl.BlockSpec((_BRH, _N), lambda i: (2 * i, 0)),
            pl.BlockSpec((_BRH, _N), lambda i: (2 * i + 1, 0)),
            pl.BlockSpec((1, _N), lambda i: (0, 0)),
        ],
        out_specs=pl.BlockSpec((2 * _BRH, 1), lambda i: (i, 0)),
        out_shape=jax.ShapeDtypeStruct((_R_TC, 1), jnp.float32),
    )(x, x, lt2)


def _sc_body(x_hbm, lt_hbm, out_hbm, buf, tab, out_v, sem0, sem1):
    cix = lax.axis_index("c")
    six = lax.axis_index("s")
    wid = six * _NC + cix
    base_row = _R_TC + wid * _RPW
    lane = lax.iota(jnp.int32, 16)

    pltpu.sync_copy(lt_hbm, tab)
    for k in range(_NPAD // 16):
        tab[pl.ds(k * 16, 16)] = jnp.exp(tab[pl.ds(k * 16, 16)])

    def issue(g, b, sem):
        pltpu.async_copy(
            x_hbm.at[pl.ds(base_row + g * _CH, _CH), :],
            buf.at[pl.ds(b * _CH, _CH), :],
            sem)

    def drain(b, sem):
        pltpu.make_async_copy(
            x_hbm.at[pl.ds(0, _CH), :],
            buf.at[pl.ds(b * _CH, _CH), :],
            sem).wait()

    def compute_group(g, b):

        def row_body(r, acc):
            row = b * _CH + r
            ms = [buf[row, pl.ds(a * 16, 16)] for a in range(4)]
            cis = [jnp.full((16,), a, jnp.int32) for a in range(4)]
            for c in range(4, _NFULL):
                a = c % 4
                v = buf[row, pl.ds(c * 16, 16)]
                gt = v > ms[a]
                ms[a] = jnp.where(gt, v, ms[a])
                cis[a] = jnp.where(gt, c, cis[a])

            def merge(m1, c1, m2, c2):
                take = jnp.logical_or(
                    m2 > m1, jnp.logical_and(m2 == m1, c2 < c1))
                return jnp.where(take, m2, m1), jnp.where(take, c2, c1)

            m, ci = merge(*merge(ms[0], cis[0], ms[1], cis[1]),
                          *merge(ms[2], cis[2], ms[3], cis[3]))
            col = ci * 16 + lane
            tl0 = buf[row, pl.ds(_TBASE, 16)]
            tl = tl0.at[lane ^ 8].get(mode="promise_in_bounds")
            gt2 = tl > m
            m = jnp.where(gt2, tl, m)
            col = jnp.where(gt2, _NFULL * 16 + lane, col)
            for s in (8, 4, 2, 1):
                p = lane ^ s
                m2 = m.at[p].get(mode="promise_in_bounds")
                c2 = col.at[p].get(mode="promise_in_bounds")
                take = jnp.logical_or(
                    m2 > m, jnp.logical_and(m2 == m, c2 < col))
                m = jnp.where(take, m2, m)
                col = jnp.where(take, c2, col)
            tv = tab[pl.ds(col[0], 16)]
            return jnp.where(lane == r, tv[0], acc)

        rowvals = lax.fori_loop(0, _CH, row_body,
                                jnp.zeros((16,), jnp.float32))
        out_v[pl.ds(g * _CH, _CH)] = rowvals

    def pair_body(p, carry):
        g0 = 2 * p
        drain(0, sem0)
        compute_group(g0, 0)
        @pl.when(g0 + 2 < _NG)
        def _():
            issue(g0 + 2, 0, sem0)
        drain(1, sem1)
        compute_group(g0 + 1, 1)
        @pl.when(g0 + 3 < _NG)
        def _():
            issue(g0 + 3, 1, sem1)
        return carry

    issue(0, 0, sem0)
    issue(1, 1, sem1)
    lax.fori_loop(0, _NG // 2, pair_body, 0)
    pltpu.sync_copy(out_v, out_hbm.at[pl.ds(wid * _RPW, _RPW)])


def _sc_part(x, lt_pad):
    mesh = plsc.VectorSubcoreMesh(core_axis_name="c", subcore_axis_name="s")
    f = functools.partial(
        pl.kernel,
        out_type=jax.ShapeDtypeStruct((_R_SC,), jnp.float32),
        mesh=mesh,
        scratch_types=[
            pltpu.VMEM((2 * _CH, _N), jnp.float32),
            pltpu.VMEM((_NPAD,), jnp.float32),
            pltpu.VMEM((_RPW,), jnp.float32),
            pltpu.SemaphoreType.DMA,
            pltpu.SemaphoreType.DMA,
        ],
    )(_sc_body)
    return f(x, lt_pad)


def kernel(x, log_tau):
    lt_pad = jnp.pad(log_tau, (0, _NPAD - _N))
    parts = []
    if _R_TC:
        parts.append(_tc_part(x, log_tau.reshape(1, _N)))
    if _R_SC:
        parts.append(_sc_part(x, lt_pad)[:, None])
    if len(parts) == 1:
        return parts[0]
    return jnp.concatenate(parts, axis=0)


# speedup vs baseline: 1.0478x; 1.0478x over previous
import functools

import jax
import jax.numpy as jnp
from jax import lax
from jax.experimental import pallas as pl
from jax.experimental.pallas import tpu as pltpu
from jax.experimental.pallas import tpu_sc as plsc

_B = 16384
_N = 1000
_NPAD = 1024

_R_TC = 12288

_BRH = 1024

_NC = 2
_NS = 16
_NW = _NC * _NS
_R_SC = _B - _R_TC
_RPW = _R_SC // _NW
_CH = 16
_NG = _RPW // _CH
_NFULL = _N // 16
_TBASE = _N - 16


def _tau_rows(xv, tab):
    m = jnp.max(xv, axis=1, keepdims=True)
    cols = jax.lax.broadcasted_iota(jnp.int32, xv.shape, 1)
    idx = jnp.min(jnp.where(xv == m, cols, _N), axis=1)
    onehot = cols == idx[:, None]
    return jnp.sum(jnp.where(onehot, tab, 0.0), axis=1)


def _tc_body(xa_ref, xb_ref, lt_ref, o_ref):
    tab = jnp.exp(lt_ref[...])
    o_ref[:_BRH, :] = _tau_rows(xa_ref[...], tab)[:, None]
    o_ref[_BRH:, :] = _tau_rows(xb_ref[...], tab)[:, None]


def _tc_part(x, lt2):
    return pl.pallas_call(
        _tc_body,
        grid=(_R_TC // (2 * _BRH),),
        in_specs=[
            pl.BlockSpec((_BRH, _N), lambda i: (2 * i, 0)),
            pl.BlockSpec((_BRH, _N), lambda i: (2 * i + 1, 0)),
            pl.BlockSpec((1, _N), lambda i: (0, 0)),
        ],
        out_specs=pl.BlockSpec((2 * _BRH, 1), lambda i: (i, 0)),
        out_shape=jax.ShapeDtypeStruct((_R_TC, 1), jnp.float32),
    )(x, x, lt2)


def _sc_body(x_hbm, lt_hbm, out_hbm, buf, tab, out_v, sem0, sem1):
    cix = lax.axis_index("c")
    six = lax.axis_index("s")
    wid = six * _NC + cix
    base_row = _R_TC + wid * _RPW
    lane = lax.iota(jnp.int32, 16)

    pltpu.sync_copy(lt_hbm, tab)
    for k in range(_NPAD // 16):
        tab[pl.ds(k * 16, 16)] = jnp.exp(tab[pl.ds(k * 16, 16)])

    def issue(g, b, sem):
        pltpu.async_copy(
            x_hbm.at[pl.ds(base_row + g * _CH, _CH), :],
            buf.at[pl.ds(b * _CH, _CH), :],
            sem)

    def drain(b, sem):
        pltpu.make_async_copy(
            x_hbm.at[pl.ds(0, _CH), :],
            buf.at[pl.ds(b * _CH, _CH), :],
            sem).wait()

    def compute_group(g, b):

        def row_body(r, acc):
            row = b * _CH + r
            ms = [buf[row, pl.ds(a * 16, 16)] for a in range(4)]
            cis = [jnp.full((16,), a, jnp.int32) for a in range(4)]
            for c in range(4, _NFULL):
                a = c % 4
                v = buf[row, pl.ds(c * 16, 16)]
                gt = v > ms[a]
                ms[a] = jnp.where(gt, v, ms[a])
                cis[a] = jnp.where(gt, c, cis[a])

            def merge(m1, c1, m2, c2):
                take = jnp.logical_or(
                    m2 > m1, jnp.logical_and(m2 == m1, c2 < c1))
                return jnp.where(take, m2, m1), jnp.where(take, c2, c1)

            m, ci = merge(*merge(ms[0], cis[0], ms[1], cis[1]),
                          *merge(ms[2], cis[2], ms[3], cis[3]))
            col = ci * 16 + lane
            tl0 = buf[row, pl.ds(_TBASE, 16)]
            tl = tl0.at[lane ^ 8].get(mode="promise_in_bounds")
            gt2 = tl > m
            m = jnp.where(gt2, tl, m)
            col = jnp.where(gt2, _NFULL * 16 + lane, col)
            for s in (8, 4, 2, 1):
                p = lane ^ s
                m2 = m.at[p].get(mode="promise_in_bounds")
                c2 = col.at[p].get(mode="promise_in_bounds")
                take = jnp.logical_or(
                    m2 > m, jnp.logical_and(m2 == m, c2 < col))
                m = jnp.where(take, m2, m)
                col = jnp.where(take, c2, col)
            tv = tab[pl.ds(col[0], 16)]
            return jnp.where(lane == r, tv[0], acc)

        rowvals = lax.fori_loop(0, _CH, row_body,
                                jnp.zeros((16,), jnp.float32))
        out_v[pl.ds(g * _CH, _CH)] = rowvals

    def pair_body(p, carry):
        g0 = 2 * p
        drain(0, sem0)
        compute_group(g0, 0)
        @pl.when(g0 + 2 < _NG)
        def _():
            issue(g0 + 2, 0, sem0)
        drain(1, sem1)
        compute_group(g0 + 1, 1)
        @pl.when(g0 + 3 < _NG)
        def _():
            issue(g0 + 3, 1, sem1)
        return carry

    issue(0, 0, sem0)
    issue(1, 1, sem1)
    lax.fori_loop(0, _NG // 2, pair_body, 0)
    pltpu.sync_copy(out_v, out_hbm.at[pl.ds(wid * _RPW, _RPW)])


def _sc_part(x, lt_pad):
    mesh = plsc.VectorSubcoreMesh(core_axis_name="c", subcore_axis_name="s")
    f = functools.partial(
        pl.kernel,
        out_type=jax.ShapeDtypeStruct((_R_SC,), jnp.float32),
        mesh=mesh,
        scratch_types=[
            pltpu.VMEM((2 * _CH, _N), jnp.float32),
            pltpu.VMEM((_NPAD,), jnp.float32),
            pltpu.VMEM((_RPW,), jnp.float32),
            pltpu.SemaphoreType.DMA,
            pltpu.SemaphoreType.DMA,
        ],
    )(_sc_body)
    return f(x, lt_pad)


def kernel(x, log_tau):
    lt_pad = jnp.pad(log_tau, (0, _NPAD - _N))
    parts = []
    if _R_TC:
        parts.append(_tc_part(x, log_tau.reshape(1, _N)))
    if _R_SC:
        parts.append(_sc_part(x, lt_pad)[:, None])
    if len(parts) == 1:
        return parts[0]
    return jnp.concatenate(parts, axis=0)


# speedup vs baseline: 1.0821x; 1.0327x over previous
import functools

import jax
import jax.numpy as jnp
from jax import lax
from jax.experimental import pallas as pl
from jax.experimental.pallas import tpu as pltpu
from jax.experimental.pallas import tpu_sc as plsc

_B = 16384
_N = 1000
_NPAD = 1024

_R_TC = 8192

_BRH = 1024

_NC = 2
_NS = 16
_NW = _NC * _NS
_R_SC = _B - _R_TC
_RPW = _R_SC // _NW
_CH = 16
_NG = _RPW // _CH
_NFULL = _N // 16
_TBASE = _N - 16


def _tau_rows(xv, tab):
    m = jnp.max(xv, axis=1, keepdims=True)
    cols = jax.lax.broadcasted_iota(jnp.int32, xv.shape, 1)
    idx = jnp.min(jnp.where(xv == m, cols, _N), axis=1)
    onehot = cols == idx[:, None]
    return jnp.sum(jnp.where(onehot, tab, 0.0), axis=1)


def _tc_body(xa_ref, xb_ref, lt_ref, o_ref):
    tab = jnp.exp(lt_ref[...])
    o_ref[:_BRH, :] = _tau_rows(xa_ref[...], tab)[:, None]
    o_ref[_BRH:, :] = _tau_rows(xb_ref[...], tab)[:, None]


def _tc_part(x, lt2):
    return pl.pallas_call(
        _tc_body,
        grid=(_R_TC // (2 * _BRH),),
        in_specs=[
            pl.BlockSpec((_BRH, _N), lambda i: (2 * i, 0)),
            pl.BlockSpec((_BRH, _N), lambda i: (2 * i + 1, 0)),
            pl.BlockSpec((1, _N), lambda i: (0, 0)),
        ],
        out_specs=pl.BlockSpec((2 * _BRH, 1), lambda i: (i, 0)),
        out_shape=jax.ShapeDtypeStruct((_R_TC, 1), jnp.float32),
    )(x, x, lt2)


def _sc_body(x_hbm, lt_hbm, out_hbm, buf, tab, out_v, sem0, sem1):
    cix = lax.axis_index("c")
    six = lax.axis_index("s")
    wid = six * _NC + cix
    base_row = _R_TC + wid * _RPW
    lane = lax.iota(jnp.int32, 16)

    pltpu.sync_copy(lt_hbm, tab)
    for k in range(_NPAD // 16):
        tab[pl.ds(k * 16, 16)] = jnp.exp(tab[pl.ds(k * 16, 16)])

    def issue(g, b, sem):
        pltpu.async_copy(
            x_hbm.at[pl.ds(base_row + g * _CH, _CH), :],
            buf.at[pl.ds(b * _CH, _CH), :],
            sem)

    def drain(b, sem):
        pltpu.make_async_copy(
            x_hbm.at[pl.ds(0, _CH), :],
            buf.at[pl.ds(b * _CH, _CH), :],
            sem).wait()

    def compute_group(g, b):

        def row_body(r, acc):
            row = b * _CH + r
            ms = [buf[row, pl.ds(a * 16, 16)] for a in range(4)]
            cis = [jnp.full((16,), a, jnp.int32) for a in range(4)]
            for c in range(4, _NFULL):
                a = c % 4
                v = buf[row, pl.ds(c * 16, 16)]
                gt = v > ms[a]
                ms[a] = jnp.where(gt, v, ms[a])
                cis[a] = jnp.where(gt, c, cis[a])

            def merge(m1, c1, m2, c2):
                take = jnp.logical_or(
                    m2 > m1, jnp.logical_and(m2 == m1, c2 < c1))
                return jnp.where(take, m2, m1), jnp.where(take, c2, c1)

            m, ci = merge(*merge(ms[0], cis[0], ms[1], cis[1]),
                          *merge(ms[2], cis[2], ms[3], cis[3]))
            col = ci * 16 + lane
            tl0 = buf[row, pl.ds(_TBASE, 16)]
            tl = tl0.at[lane ^ 8].get(mode="promise_in_bounds")
            gt2 = tl > m
            m = jnp.where(gt2, tl, m)
            col = jnp.where(gt2, _NFULL * 16 + lane, col)
            for s in (8, 4, 2, 1):
                p = lane ^ s
                m2 = m.at[p].get(mode="promise_in_bounds")
                c2 = col.at[p].get(mode="promise_in_bounds")
                take = jnp.logical_or(
                    m2 > m, jnp.logical_and(m2 == m, c2 < col))
                m = jnp.where(take, m2, m)
                col = jnp.where(take, c2, col)
            tv = tab[pl.ds(col[0], 16)]
            return jnp.where(lane == r, tv[0], acc)

        rowvals = lax.fori_loop(0, _CH, row_body,
                                jnp.zeros((16,), jnp.float32))
        out_v[pl.ds(g * _CH, _CH)] = rowvals

    def pair_body(p, carry):
        g0 = 2 * p
        drain(0, sem0)
        compute_group(g0, 0)
        @pl.when(g0 + 2 < _NG)
        def _():
            issue(g0 + 2, 0, sem0)
        drain(1, sem1)
        compute_group(g0 + 1, 1)
        @pl.when(g0 + 3 < _NG)
        def _():
            issue(g0 + 3, 1, sem1)
        return carry

    issue(0, 0, sem0)
    issue(1, 1, sem1)
    lax.fori_loop(0, _NG // 2, pair_body, 0)
    pltpu.sync_copy(out_v, out_hbm.at[pl.ds(wid * _RPW, _RPW)])


def _sc_part(x, lt_pad):
    mesh = plsc.VectorSubcoreMesh(core_axis_name="c", subcore_axis_name="s")
    f = functools.partial(
        pl.kernel,
        out_type=jax.ShapeDtypeStruct((_R_SC,), jnp.float32),
        mesh=mesh,
        scratch_types=[
            pltpu.VMEM((2 * _CH, _N), jnp.float32),
            pltpu.VMEM((_NPAD,), jnp.float32),
            pltpu.VMEM((_RPW,), jnp.float32),
            pltpu.SemaphoreType.DMA,
            pltpu.SemaphoreType.DMA,
        ],
    )(_sc_body)
    return f(x, lt_pad)


def kernel(x, log_tau):
    lt_pad = jnp.pad(log_tau, (0, _NPAD - _N))
    parts = []
    if _R_TC:
        parts.append(_tc_part(x, log_tau.reshape(1, _N)))
    if _R_SC:
        parts.append(_sc_part(x, lt_pad)[:, None])
    if len(parts) == 1:
        return parts[0]
    return jnp.concatenate(parts, axis=0)


# speedup vs baseline: 1.0927x; 1.0098x over previous
import functools

import jax
import jax.numpy as jnp
from jax import lax
from jax.experimental import pallas as pl
from jax.experimental.pallas import tpu as pltpu
from jax.experimental.pallas import tpu_sc as plsc

_B = 16384
_N = 1000
_NPAD = 1024

_R_TC = 9216

_BRH = 512

_NC = 2
_NS = 16
_NW = _NC * _NS
_R_SC = _B - _R_TC
_RPW = _R_SC // _NW
_CH = 16
_NG = _RPW // _CH
_NFULL = _N // 16
_TBASE = _N - 16


def _tau_rows(xv, tab):
    m = jnp.max(xv, axis=1, keepdims=True)
    cols = jax.lax.broadcasted_iota(jnp.int32, xv.shape, 1)
    idx = jnp.min(jnp.where(xv == m, cols, _N), axis=1)
    onehot = cols == idx[:, None]
    return jnp.sum(jnp.where(onehot, tab, 0.0), axis=1)


def _tc_body(xa_ref, xb_ref, lt_ref, o_ref):
    tab = jnp.exp(lt_ref[...])
    o_ref[:_BRH, :] = _tau_rows(xa_ref[...], tab)[:, None]
    o_ref[_BRH:, :] = _tau_rows(xb_ref[...], tab)[:, None]


def _tc_part(x, lt2):
    return pl.pallas_call(
        _tc_body,
        grid=(_R_TC // (2 * _BRH),),
        in_specs=[
            pl.BlockSpec((_BRH, _N), lambda i: (2 * i, 0)),
            pl.BlockSpec((_BRH, _N), lambda i: (2 * i + 1, 0)),
            pl.BlockSpec((1, _N), lambda i: (0, 0)),
        ],
        out_specs=pl.BlockSpec((2 * _BRH, 1), lambda i: (i, 0)),
        out_shape=jax.ShapeDtypeStruct((_R_TC, 1), jnp.float32),
    )(x, x, lt2)


def _sc_body(x_hbm, lt_hbm, out_hbm, buf, tab, out_v, sem0, sem1):
    cix = lax.axis_index("c")
    six = lax.axis_index("s")
    wid = six * _NC + cix
    base_row = _R_TC + wid * _RPW
    lane = lax.iota(jnp.int32, 16)

    pltpu.sync_copy(lt_hbm, tab)
    for k in range(_NPAD // 16):
        tab[pl.ds(k * 16, 16)] = jnp.exp(tab[pl.ds(k * 16, 16)])

    def issue(g, b, sem):
        pltpu.async_copy(
            x_hbm.at[pl.ds(base_row + g * _CH, _CH), :],
            buf.at[pl.ds(b * _CH, _CH), :],
            sem)

    def drain(b, sem):
        pltpu.make_async_copy(
            x_hbm.at[pl.ds(0, _CH), :],
            buf.at[pl.ds(b * _CH, _CH), :],
            sem).wait()

    def compute_group(g, b):

        def row_body(r, acc):
            row = b * _CH + r
            ms = [buf[row, pl.ds(a * 16, 16)] for a in range(4)]
            cis = [jnp.full((16,), a, jnp.int32) for a in range(4)]
            for c in range(4, _NFULL):
                a = c % 4
                v = buf[row, pl.ds(c * 16, 16)]
                gt = v > ms[a]
                ms[a] = jnp.where(gt, v, ms[a])
                cis[a] = jnp.where(gt, c, cis[a])

            def merge(m1, c1, m2, c2):
                take = jnp.logical_or(
                    m2 > m1, jnp.logical_and(m2 == m1, c2 < c1))
                return jnp.where(take, m2, m1), jnp.where(take, c2, c1)

            m, ci = merge(*merge(ms[0], cis[0], ms[1], cis[1]),
                          *merge(ms[2], cis[2], ms[3], cis[3]))
            col = ci * 16 + lane
            tl0 = buf[row, pl.ds(_TBASE, 16)]
            tl = tl0.at[lane ^ 8].get(mode="promise_in_bounds")
            gt2 = tl > m
            m = jnp.where(gt2, tl, m)
            col = jnp.where(gt2, _NFULL * 16 + lane, col)
            for s in (8, 4, 2, 1):
                p = lane ^ s
                m2 = m.at[p].get(mode="promise_in_bounds")
                c2 = col.at[p].get(mode="promise_in_bounds")
                take = jnp.logical_or(
                    m2 > m, jnp.logical_and(m2 == m, c2 < col))
                m = jnp.where(take, m2, m)
                col = jnp.where(take, c2, col)
            tv = tab[pl.ds(col[0], 16)]
            return jnp.where(lane == r, tv[0], acc)

        rowvals = lax.fori_loop(0, _CH, row_body,
                                jnp.zeros((16,), jnp.float32))
        out_v[pl.ds(g * _CH, _CH)] = rowvals

    def pair_body(p, carry):
        g0 = 2 * p
        drain(0, sem0)
        compute_group(g0, 0)
        @pl.when(g0 + 2 < _NG)
        def _():
            issue(g0 + 2, 0, sem0)
        drain(1, sem1)
        compute_group(g0 + 1, 1)
        @pl.when(g0 + 3 < _NG)
        def _():
            issue(g0 + 3, 1, sem1)
        return carry

    issue(0, 0, sem0)
    issue(1, 1, sem1)
    lax.fori_loop(0, _NG // 2, pair_body, 0)
    pltpu.sync_copy(out_v, out_hbm.at[pl.ds(wid * _RPW, _RPW)])


def _sc_part(x, lt_pad):
    mesh = plsc.VectorSubcoreMesh(core_axis_name="c", subcore_axis_name="s")
    f = functools.partial(
        pl.kernel,
        out_type=jax.ShapeDtypeStruct((_R_SC,), jnp.float32),
        mesh=mesh,
        scratch_types=[
            pltpu.VMEM((2 * _CH, _N), jnp.float32),
            pltpu.VMEM((_NPAD,), jnp.float32),
            pltpu.VMEM((_RPW,), jnp.float32),
            pltpu.SemaphoreType.DMA,
            pltpu.SemaphoreType.DMA,
        ],
    )(_sc_body)
    return f(x, lt_pad)


def kernel(x, log_tau):
    lt_pad = jnp.pad(log_tau, (0, _NPAD - _N))
    parts = []
    if _R_TC:
        parts.append(_tc_part(x, log_tau.reshape(1, _N)))
    if _R_SC:
        parts.append(_sc_part(x, lt_pad)[:, None])
    if len(parts) == 1:
        return parts[0]
    return jnp.concatenate(parts, axis=0)
